# baseline (speedup 1.0000x reference)
import functools

import jax
import jax.numpy as jnp
from jax import lax
from jax.experimental import pallas as pl
from jax.experimental.pallas import tpu as pltpu
from jax.experimental.pallas import tpu_sc as plsc

_B = 16384
_NS = 64
_NP = _NS + 1
_D = 32
_TEMPERATURE = 0.07
_BASE_TEMPERATURE = 0.07
_EPS = 1.0
_CLAMP_LOW = 0.0001

_L = 16
_NC = 2
_NSUB = 16
_NW = _NC * _NSUB
_ROWS_PER_W = _B // _NW
_R = 16
_CHUNKS = _ROWS_PER_W // _R
_G = _NS // _L
_W = _D // 2


def _neigh_inds():
    neg = jax.random.randint(jax.random.key(1), (_B, _NS), 0, 2 * _B).astype(jnp.int32)
    pos = jnp.arange(_B, 2 * _B, dtype=jnp.int32)[:, None]
    return jnp.concatenate([pos, neg], axis=1).reshape(-1)


def _neigh_flat():
    try:
        with jax.ensure_compile_time_eval():
            return _neigh_inds()
    except Exception:
        return _neigh_inds()


_LN2 = 0.6931471805599453


def _log16(x):
    y = plsc.bitcast(x, jnp.int32)
    e = (y >> 23) - 127
    m = plsc.bitcast((y & 0x007FFFFF) | 0x3F800000, jnp.float32)
    s = (m - 1.0) / (m + 1.0)
    s2 = s * s
    p = 1.0 + s2 * (1.0 / 3.0 + s2 * (1.0 / 5.0 + s2 * (1.0 / 7.0)))
    return e.astype(jnp.float32) * _LN2 + 2.0 * s * p

@functools.lru_cache(maxsize=1)
def _build_sc_ratio():
    mesh = plsc.VectorSubcoreMesh(core_axis_name="c", subcore_axis_name="s")

    @functools.partial(
        pl.kernel,
        out_type=jax.ShapeDtypeStruct((_NW, _L), jnp.float32),
        mesh=mesh,
        scratch_types=[
            pltpu.VMEM((_ROWS_PER_W * _NP,), jnp.int32),
            pltpu.VMEM((_R * _NP, _D), jnp.float32),
            pltpu.VMEM((_R * _NP, _D), jnp.float32),
            pltpu.VMEM((_ROWS_PER_W, _D), jnp.float32),
            pltpu.VMEM((_L,), jnp.float32),
            pltpu.SemaphoreType.DMA,
            pltpu.SemaphoreType.DMA,
        ],
        compiler_params=pltpu.CompilerParams(
            use_tc_tiling_on_sc=False, needs_layout_passes=False
        ),
    )
    def _sc_ratio(feat_hbm, nidx_hbm, out_hbm, idx_v, rows0_v, rows1_v,
                  orig_v, part_v, sem0, sem1):
        wid = lax.axis_index("s") * _NC + lax.axis_index("c")
        iota = lax.iota(jnp.int32, _L)
        rbase = wid * _ROWS_PER_W

        pltpu.sync_copy(nidx_hbm.at[pl.ds(rbase * _NP, _ROWS_PER_W * _NP)], idx_v)
        pltpu.sync_copy(feat_hbm.at[pl.ds(rbase, _ROWS_PER_W)], orig_v)

        def start_gather(c, rows_v, sem):
            src = feat_hbm.at[idx_v.at[pl.ds(c * (_R * _NP), _R * _NP)]]
            pltpu.async_copy(src, rows_v, sem)

        def wait_gather(c, rows_v, sem):
            src = feat_hbm.at[idx_v.at[pl.ds(c * (_R * _NP), _R * _NP)]]
            pltpu.make_async_copy(src, rows_v, sem).wait()

        def compute_chunk(c, rows_v):
            orow = c * _R + iota
            acc = jnp.zeros((_L,), jnp.float32)
            prow = iota * _NP
            for k in range(_D):
                dv = (iota + k) & (_D - 1)
                o = plsc.load_gather(orig_v, [orow, dv])
                nv = plsc.load_gather(rows_v, [prow, dv])
                t = o - nv
                acc = acc + t * t
            pos = jnp.minimum(jnp.maximum(1.0 / (_EPS + acc), _CLAMP_LOW), 1.0)

            def row_body(r, svec):
                accs = [jnp.zeros((_L,), jnp.float32) for _ in range(_G)]
                nbase = r * _NP + 1
                rv = jnp.full((_L,), c * _R + r, jnp.int32)
                for k in range(_D):
                    dv = (iota + k) & (_D - 1)
                    ov = plsc.load_gather(orig_v, [rv, dv])
                    for g in range(_G):
                        ridx = jnp.full((_L,), nbase + g * _L, jnp.int32) + iota
                        nv = plsc.load_gather(rows_v, [ridx, dv])
                        t = ov - nv
                        accs[g] = accs[g] + t * t
                pc = jnp.zeros((_L,), jnp.float32)
                for g in range(_G):
                    p = 1.0 / (_EPS + accs[g])
                    pc = pc + jnp.minimum(jnp.maximum(p, _CLAMP_LOW), 1.0)
                return jnp.where(iota == r, jnp.sum(pc), svec)

            svec = plsc.parallel_loop(
                0, _R, unroll=2, carry=jnp.zeros((_L,), jnp.float32)
            )(row_body)
            return _log16(svec / pos)

        start_gather(0, rows0_v, sem0)

        def body(i, acc):
            c0 = 2 * i
            start_gather(c0 + 1, rows1_v, sem1)
            wait_gather(c0, rows0_v, sem0)
            acc = acc + compute_chunk(c0, rows0_v)

            @pl.when(i < _CHUNKS // 2 - 1)
            def _():
                start_gather(c0 + 2, rows0_v, sem0)

            wait_gather(c0 + 1, rows1_v, sem1)
            acc = acc + compute_chunk(c0 + 1, rows1_v)
            return acc

        acc = lax.fori_loop(
            0, _CHUNKS // 2, body, jnp.zeros((_L,), jnp.float32)
        )
        part_v[...] = acc
        pltpu.sync_copy(part_v, out_hbm.at[wid])

    return _sc_ratio


def kernel(features):
    nidx = _neigh_flat()
    parts = _build_sc_ratio()(features, nidx)
    return (_TEMPERATURE / _BASE_TEMPERATURE) * jnp.sum(parts)


# speedup vs baseline: 1.0360x; 1.0360x over previous
import functools

import jax
import jax.numpy as jnp
from jax import lax
from jax.experimental import pallas as pl
from jax.experimental.pallas import tpu as pltpu
from jax.experimental.pallas import tpu_sc as plsc

_B = 16384
_NS = 64
_NP = _NS + 1
_D = 32
_TEMPERATURE = 0.07
_BASE_TEMPERATURE = 0.07
_EPS = 1.0
_CLAMP_LOW = 0.0001

_L = 16
_NC = 2
_NSUB = 16
_NW = _NC * _NSUB
_ROWS_PER_W = _B // _NW
_R = 16
_CHUNKS = _ROWS_PER_W // _R
_G = _NS // _L
_W = _D // 2


def _neigh_inds():
    neg = jax.random.randint(jax.random.key(1), (_B, _NS), 0, 2 * _B).astype(jnp.int32)
    pos = jnp.arange(_B, 2 * _B, dtype=jnp.int32)[:, None]
    return jnp.concatenate([pos, neg], axis=1).reshape(-1)


def _neigh_flat():
    try:
        with jax.ensure_compile_time_eval():
            return _neigh_inds()
    except Exception:
        return _neigh_inds()


_LN2 = 0.6931471805599453


def _log16(x):
    y = plsc.bitcast(x, jnp.int32)
    e = (y >> 23) - 127
    m = plsc.bitcast((y & 0x007FFFFF) | 0x3F800000, jnp.float32)
    s = (m - 1.0) / (m + 1.0)
    s2 = s * s
    p = 1.0 + s2 * (1.0 / 3.0 + s2 * (1.0 / 5.0 + s2 * (1.0 / 7.0)))
    return e.astype(jnp.float32) * _LN2 + 2.0 * s * p

@functools.lru_cache(maxsize=1)
def _build_sc_ratio():
    mesh = plsc.VectorSubcoreMesh(core_axis_name="c", subcore_axis_name="s")

    @functools.partial(
        pl.kernel,
        out_type=jax.ShapeDtypeStruct((_NW, _L), jnp.float32),
        mesh=mesh,
        scratch_types=[
            pltpu.VMEM((_ROWS_PER_W * _NP,), jnp.int32),
            pltpu.VMEM((_R * _NP, _W), jnp.int32),
            pltpu.VMEM((_R * _NP, _W), jnp.int32),
            pltpu.VMEM((_ROWS_PER_W, _W), jnp.int32),
            pltpu.VMEM((_L,), jnp.float32),
            pltpu.SemaphoreType.DMA,
            pltpu.SemaphoreType.DMA,
        ],
        compiler_params=pltpu.CompilerParams(
            use_tc_tiling_on_sc=False, needs_layout_passes=False
        ),
    )
    def _sc_ratio(feat_hbm, nidx_hbm, out_hbm, idx_v, rows0_v, rows1_v,
                  orig_v, part_v, sem0, sem1):
        wid = lax.axis_index("s") * _NC + lax.axis_index("c")
        iota = lax.iota(jnp.int32, _L)
        rbase = wid * _ROWS_PER_W

        pltpu.sync_copy(nidx_hbm.at[pl.ds(rbase * _NP, _ROWS_PER_W * _NP)], idx_v)
        pltpu.sync_copy(feat_hbm.at[pl.ds(rbase, _ROWS_PER_W)], orig_v)

        def start_gather(c, rows_v, sem):
            src = feat_hbm.at[idx_v.at[pl.ds(c * (_R * _NP), _R * _NP)]]
            pltpu.async_copy(src, rows_v, sem)

        def wait_gather(c, rows_v, sem):
            src = feat_hbm.at[idx_v.at[pl.ds(c * (_R * _NP), _R * _NP)]]
            pltpu.make_async_copy(src, rows_v, sem).wait()

        def compute_chunk(c, rows_v):
            orow = c * _R + iota
            acca = jnp.zeros((_L,), jnp.float32)
            accb = jnp.zeros((_L,), jnp.float32)
            prow = iota * _NP
            for k in range(_W):
                dv = (iota + k) & (_W - 1)
                o = plsc.bitcast(plsc.load_gather(orig_v, [orow, dv]), jnp.bfloat16)
                nv = plsc.bitcast(plsc.load_gather(rows_v, [prow, dv]), jnp.bfloat16)
                t = o - nv
                a, b = plsc.unpack(t * t, format=plsc.PackFormat.INTERLEAVED)
                acca = acca + a
                accb = accb + b
            pos = jnp.minimum(
                jnp.maximum(1.0 / (_EPS + (acca + accb)), _CLAMP_LOW), 1.0
            )

            def row_body(r, svec):
                accs = [[jnp.zeros((_L,), jnp.float32)] * 2 for _ in range(_G)]
                nbase = r * _NP + 1
                rv = jnp.full((_L,), c * _R + r, jnp.int32)
                for k in range(_W):
                    dv = (iota + k) & (_W - 1)
                    ov = plsc.bitcast(plsc.load_gather(orig_v, [rv, dv]), jnp.bfloat16)
                    for g in range(_G):
                        ridx = jnp.full((_L,), nbase + g * _L, jnp.int32) + iota
                        nv = plsc.bitcast(
                            plsc.load_gather(rows_v, [ridx, dv]), jnp.bfloat16
                        )
                        t = ov - nv
                        a, b = plsc.unpack(t * t, format=plsc.PackFormat.INTERLEAVED)
                        accs[g] = [accs[g][0] + a, accs[g][1] + b]
                pc = jnp.zeros((_L,), jnp.float32)
                for g in range(_G):
                    p = 1.0 / (_EPS + (accs[g][0] + accs[g][1]))
                    pc = pc + jnp.minimum(jnp.maximum(p, _CLAMP_LOW), 1.0)
                return jnp.where(iota == r, jnp.sum(pc), svec)

            svec = plsc.parallel_loop(
                0, _R, unroll=2, carry=jnp.zeros((_L,), jnp.float32)
            )(row_body)
            return _log16(svec / pos)

        start_gather(0, rows0_v, sem0)

        def body(i, acc):
            c0 = 2 * i
            start_gather(c0 + 1, rows1_v, sem1)
            wait_gather(c0, rows0_v, sem0)
            acc = acc + compute_chunk(c0, rows0_v)

            @pl.when(i < _CHUNKS // 2 - 1)
            def _():
                start_gather(c0 + 2, rows0_v, sem0)

            wait_gather(c0 + 1, rows1_v, sem1)
            acc = acc + compute_chunk(c0 + 1, rows1_v)
            return acc

        acc = lax.fori_loop(
            0, _CHUNKS // 2, body, jnp.zeros((_L,), jnp.float32)
        )
        part_v[...] = acc
        pltpu.sync_copy(part_v, out_hbm.at[wid])

    return _sc_ratio


def kernel(features):
    nidx = _neigh_flat()
    fb = features.astype(jnp.bfloat16).reshape(_B * 2, _W, 2)
    packed = jax.lax.bitcast_convert_type(fb, jnp.int32)
    parts = _build_sc_ratio()(packed, nidx)
    return (_TEMPERATURE / _BASE_TEMPERATURE) * jnp.sum(parts)
